# baseline (speedup 1.0000x reference)
import functools
import math

import jax
import jax.numpy as jnp
from jax import lax
from jax.experimental import pallas as pl
from jax.experimental.pallas import tpu as pltpu
from jax.experimental.pallas import tpu_sc as plsc

B, S, H, SKV, G, D, T, K = 32, 1, 16, 8192, 1, 128, 64, 1024
DT = D + T


def _sc_counts(idx_flat):
    info = plsc.get_sparse_core_info()
    nc = info.num_cores

    mesh = plsc.VectorSubcoreMesh(core_axis_name="c", subcore_axis_name="s")

    @functools.partial(
        pl.kernel,
        mesh=mesh,
        compiler_params=pltpu.CompilerParams(needs_layout_passes=False),
        out_type=jax.ShapeDtypeStruct((B, 1, SKV), jnp.float32),
        scratch_types=[
            pltpu.VMEM((K,), jnp.int32),
            pltpu.VMEM((SKV,), jnp.float32),
        ],
    )
    def counts_kernel(idx_hbm, out_hbm, idx_v, cnt_v):
        wid = lax.axis_index("s") * nc + lax.axis_index("c")
        pltpu.sync_copy(idx_hbm.at[pl.ds(wid * K, K)], idx_v)

        zeros = jnp.zeros((16,), jnp.float32)

        def zero_body(i, _):
            cnt_v[pl.ds(i * 16, 16)] = zeros
            return 0

        lax.fori_loop(0, SKV // 16, zero_body, 0)

        ones = jnp.ones((16,), jnp.float32)

        def acc_body(i, _):
            idx16 = idx_v[pl.ds(i * 16, 16)]
            plsc.addupdate_scatter(cnt_v, [idx16], ones)
            return 0

        lax.fori_loop(0, K // 16, acc_body, 0)

        pltpu.sync_copy(cnt_v, out_hbm.at[wid, 0])

    return counts_kernel(idx_flat)


BPB = 4


def _attn_body(q_ref, kvt_ref, cnt_ref, o_ref):
    sm_scale = 1.0 / math.sqrt(DT)
    for i in range(BPB):
        qb = q_ref[i]
        kvt = kvt_ref[i]
        c = cnt_ref[i]
        logits = lax.dot_general(
            qb, kvt, (((1,), (0,)), ((), ())),
            preferred_element_type=jnp.float32) * sm_scale
        bias = jnp.where(c > 0.0, jnp.log(c), -jnp.inf)
        logits = logits + bias
        m = jnp.max(logits, axis=1, keepdims=True)
        p = jnp.exp(logits - m)
        s = jnp.sum(p, axis=1, keepdims=True)
        o = lax.dot_general(
            p, kvt[:D, :], (((1,), (1,)), ((), ())),
            preferred_element_type=jnp.float32)
        o_ref[i] = o / s


def kernel(q, kv, indices):
    idx_flat = indices.reshape(B * K)
    counts = _sc_counts(idx_flat)

    kvt = jnp.transpose(kv, (0, 3, 2, 1)).reshape(B, DT, SKV)

    out = pl.pallas_call(
        _attn_body,
        grid=(B // BPB,),
        in_specs=[
            pl.BlockSpec((BPB, H, DT), lambda b: (b, 0, 0)),
            pl.BlockSpec((BPB, DT, SKV), lambda b: (b, 0, 0)),
            pl.BlockSpec((BPB, 1, SKV), lambda b: (b, 0, 0)),
        ],
        out_specs=pl.BlockSpec((BPB, H, D), lambda b: (b, 0, 0)),
        out_shape=jax.ShapeDtypeStruct((B, H, D), jnp.float32),
        compiler_params=pltpu.CompilerParams(
            vmem_limit_bytes=100 * 1024 * 1024),
    )(q.reshape(B, H, DT), kvt, counts)
    return out.reshape(B, S, H, D)


# speedup vs baseline: 1.0420x; 1.0420x over previous
import functools
import math

import jax
import jax.numpy as jnp
from jax import lax
from jax.experimental import pallas as pl
from jax.experimental.pallas import tpu as pltpu
from jax.experimental.pallas import tpu_sc as plsc

B, S, H, SKV, G, D, T, K = 32, 1, 16, 8192, 1, 128, 64, 1024
DT = D + T


def _sc_counts(idx_flat):
    info = plsc.get_sparse_core_info()
    nc = info.num_cores

    mesh = plsc.VectorSubcoreMesh(core_axis_name="c", subcore_axis_name="s")

    @functools.partial(
        pl.kernel,
        mesh=mesh,
        compiler_params=pltpu.CompilerParams(needs_layout_passes=False),
        out_type=jax.ShapeDtypeStruct((B, 1, SKV), jnp.float32),
        scratch_types=[
            pltpu.VMEM((K,), jnp.int32),
            pltpu.VMEM((SKV,), jnp.float32),
        ],
    )
    def counts_kernel(idx_hbm, out_hbm, idx_v, cnt_v):
        wid = lax.axis_index("s") * nc + lax.axis_index("c")
        pltpu.sync_copy(idx_hbm.at[pl.ds(wid * K, K)], idx_v)

        zeros = jnp.zeros((16,), jnp.float32)

        def zero_body(i, _):
            cnt_v[pl.ds(i * 16, 16)] = zeros
            return 0

        lax.fori_loop(0, SKV // 16, zero_body, 0)

        ones = jnp.ones((16,), jnp.float32)

        def acc_body(i, _):
            idx16 = idx_v[pl.ds(i * 16, 16)]
            plsc.addupdate_scatter(cnt_v, [idx16], ones)
            return 0

        lax.fori_loop(0, K // 16, acc_body, 0)

        pltpu.sync_copy(cnt_v, out_hbm.at[wid, 0])

    return counts_kernel(idx_flat)


BPB = 2


def _attn_body(q_ref, kvt_ref, cnt_ref, o_ref):
    sm_scale = 1.0 / math.sqrt(DT)
    for i in range(BPB):
        qb = q_ref[i]
        kvt = kvt_ref[i]
        c = cnt_ref[i]
        logits = lax.dot_general(
            qb, kvt, (((1,), (0,)), ((), ())),
            preferred_element_type=jnp.float32) * sm_scale
        bias = jnp.where(c > 0.0, jnp.log(c), -jnp.inf)
        logits = logits + bias
        m = jnp.max(logits, axis=1, keepdims=True)
        p = jnp.exp(logits - m)
        s = jnp.sum(p, axis=1, keepdims=True)
        o = lax.dot_general(
            p, kvt[:D, :], (((1,), (1,)), ((), ())),
            preferred_element_type=jnp.float32)
        o_ref[i] = o / s


def kernel(q, kv, indices):
    idx_flat = indices.reshape(B * K)
    counts = _sc_counts(idx_flat)

    kvt = jnp.transpose(kv, (0, 3, 2, 1)).reshape(B, DT, SKV)

    out = pl.pallas_call(
        _attn_body,
        grid=(B // BPB,),
        in_specs=[
            pl.BlockSpec((BPB, H, DT), lambda b: (b, 0, 0)),
            pl.BlockSpec((BPB, DT, SKV), lambda b: (b, 0, 0)),
            pl.BlockSpec((BPB, 1, SKV), lambda b: (b, 0, 0)),
        ],
        out_specs=pl.BlockSpec((BPB, H, D), lambda b: (b, 0, 0)),
        out_shape=jax.ShapeDtypeStruct((B, H, D), jnp.float32),
        compiler_params=pltpu.CompilerParams(
            vmem_limit_bytes=100 * 1024 * 1024),
    )(q.reshape(B, H, DT), kvt, counts)
    return out.reshape(B, S, H, D)


# speedup vs baseline: 1.0654x; 1.0225x over previous
import functools
import math

import jax
import jax.numpy as jnp
from jax import lax
from jax.experimental import pallas as pl
from jax.experimental.pallas import tpu as pltpu
from jax.experimental.pallas import tpu_sc as plsc

B, S, H, SKV, G, D, T, K = 32, 1, 16, 8192, 1, 128, 64, 1024
DT = D + T


def _sc_counts(idx_flat):
    info = plsc.get_sparse_core_info()
    nc = info.num_cores

    mesh = plsc.VectorSubcoreMesh(core_axis_name="c", subcore_axis_name="s")

    @functools.partial(
        pl.kernel,
        mesh=mesh,
        compiler_params=pltpu.CompilerParams(needs_layout_passes=False),
        out_type=jax.ShapeDtypeStruct((B, 1, SKV), jnp.float32),
        scratch_types=[
            pltpu.VMEM((K,), jnp.int32),
            pltpu.VMEM((SKV,), jnp.float32),
        ],
    )
    def counts_kernel(idx_hbm, out_hbm, idx_v, cnt_v):
        wid = lax.axis_index("s") * nc + lax.axis_index("c")
        pltpu.sync_copy(idx_hbm.at[pl.ds(wid * K, K)], idx_v)

        zeros = jnp.zeros((16,), jnp.float32)

        def zero_body(i, _):
            for u in range(8):
                cnt_v[pl.ds(i * 128 + u * 16, 16)] = zeros
            return 0

        lax.fori_loop(0, SKV // 128, zero_body, 0)

        ones = jnp.ones((16,), jnp.float32)

        def acc_body(i, _):
            idx16 = idx_v[pl.ds(i * 16, 16)]
            plsc.addupdate_scatter(cnt_v, [idx16], ones)
            return 0

        lax.fori_loop(0, K // 16, acc_body, 0)

        pltpu.sync_copy(cnt_v, out_hbm.at[wid, 0])

    return counts_kernel(idx_flat)


BPB = 2


def _attn_body(q_ref, kvt_ref, cnt_ref, o_ref):
    sm_scale = 1.0 / math.sqrt(DT)
    for i in range(BPB):
        qb = q_ref[i]
        kvt = kvt_ref[i]
        c = cnt_ref[i]
        logits = lax.dot_general(
            qb, kvt, (((1,), (0,)), ((), ())),
            preferred_element_type=jnp.float32) * sm_scale
        bias = jnp.where(c > 0.0, jnp.log(c), -jnp.inf)
        logits = logits + bias
        m = jnp.max(logits, axis=1, keepdims=True)
        p = jnp.exp(logits - m)
        s = jnp.sum(p, axis=1, keepdims=True)
        o = lax.dot_general(
            p, kvt[:D, :], (((1,), (1,)), ((), ())),
            preferred_element_type=jnp.float32)
        o_ref[i] = o / s


def kernel(q, kv, indices):
    idx_flat = indices.reshape(B * K)
    counts = _sc_counts(idx_flat)

    kvt = jnp.transpose(kv, (0, 3, 2, 1)).reshape(B, DT, SKV)

    out = pl.pallas_call(
        _attn_body,
        grid=(B // BPB,),
        in_specs=[
            pl.BlockSpec((BPB, H, DT), lambda b: (b, 0, 0)),
            pl.BlockSpec((BPB, DT, SKV), lambda b: (b, 0, 0)),
            pl.BlockSpec((BPB, 1, SKV), lambda b: (b, 0, 0)),
        ],
        out_specs=pl.BlockSpec((BPB, H, D), lambda b: (b, 0, 0)),
        out_shape=jax.ShapeDtypeStruct((B, H, D), jnp.float32),
        compiler_params=pltpu.CompilerParams(
            vmem_limit_bytes=100 * 1024 * 1024),
    )(q.reshape(B, H, DT), kvt, counts)
    return out.reshape(B, S, H, D)
